# baseline (speedup 1.0000x reference)
import jax
import jax.numpy as jnp
from jax.experimental import pallas as pl
from jax.experimental.pallas import tpu as pltpu

E = 64
K = 8
R = 16
TEMP = 0.5
EPS = 1e-6
TILE = 1024


def _phase1_kernel(x_ref, ws_ref, h_ref, pmax_ref):
    h = jax.lax.dot_general(
        x_ref[:], ws_ref[:],
        dimension_numbers=(((1,), (1,)), ((), ())),
        preferred_element_type=jnp.float32,
    )
    h_ref[:] = h
    pmax_ref[:] = jnp.max(jnp.abs(h)).reshape(1, 1, 1)


def _phase2_kernel(x_ref, h_ref, pmax_ref, w_ref, a_ref, bmb_ref,
                   limes_ref, out_ref):
    x = x_ref[:].astype(jnp.bfloat16)
    base = jax.lax.dot_general(
        x, w_ref[:],
        dimension_numbers=(((1,), (1,)), ((), ())),
        preferred_element_type=jnp.float32,
    )

    scale = jnp.maximum(jnp.max(pmax_ref[:]), EPS)
    inv = 1.0 / (scale * TEMP)
    logits = h_ref[:] * inv
    m = jnp.max(logits, axis=-1, keepdims=True)
    e = jnp.exp(logits - m)
    masked = e
    for _ in range(K):
        cur = jnp.max(masked, axis=-1, keepdims=True)
        masked = jnp.where(masked == cur, -1.0, masked)
    sel_e = jnp.where(masked < 0.0, e, 0.0)
    ssum = jnp.sum(sel_e, axis=-1, keepdims=True)
    p_mix = jnp.dot(sel_e, limes_ref[:],
                    preferred_element_type=jnp.float32) / ssum

    u = jnp.dot(x, a_ref[:], preferred_element_type=jnp.float32)
    mod = (u * p_mix).astype(jnp.bfloat16)
    mod1 = jnp.concatenate(
        [mod, jnp.ones((mod.shape[0], 1), jnp.bfloat16)], axis=1)
    delta = jnp.dot(mod1, bmb_ref[:],
                    preferred_element_type=jnp.float32)
    out_ref[:] = base + delta


def kernel(x, W, b, A, Bm, LiMEs):
    B, T, d_in = x.shape
    d_out = W.shape[0]
    n_tok = B * T
    nt = n_tok // TILE
    x2 = x.reshape(n_tok, d_in)
    W_bf = W.astype(jnp.bfloat16)
    A_bf = A.astype(jnp.bfloat16)
    Bmb_bf = jnp.concatenate([Bm, b[None, :]], axis=0).astype(jnp.bfloat16)

    h, pmax = pl.pallas_call(
        _phase1_kernel,
        grid=(nt,),
        in_specs=[
            pl.BlockSpec((TILE, d_in), lambda i: (i, 0)),
            pl.BlockSpec((E, d_in), lambda i: (0, 0)),
        ],
        out_specs=[
            pl.BlockSpec((TILE, E), lambda i: (i, 0)),
            pl.BlockSpec((1, 1, 1), lambda i: (i, 0, 0)),
        ],
        out_shape=[
            jax.ShapeDtypeStruct((n_tok, E), jnp.float32),
            jax.ShapeDtypeStruct((nt, 1, 1), jnp.float32),
        ],
        compiler_params=pltpu.CompilerParams(
            dimension_semantics=("parallel",)),
    )(x2, W)

    out = pl.pallas_call(
        _phase2_kernel,
        grid=(nt,),
        in_specs=[
            pl.BlockSpec((TILE, d_in), lambda i: (i, 0)),
            pl.BlockSpec((TILE, E), lambda i: (i, 0)),
            pl.BlockSpec((nt, 1, 1), lambda i: (0, 0, 0)),
            pl.BlockSpec((d_out, d_in), lambda i: (0, 0)),
            pl.BlockSpec((d_in, R), lambda i: (0, 0)),
            pl.BlockSpec((R + 1, d_out), lambda i: (0, 0)),
            pl.BlockSpec((E, R), lambda i: (0, 0)),
        ],
        out_specs=pl.BlockSpec((TILE, d_out), lambda i: (i, 0)),
        out_shape=jax.ShapeDtypeStruct((n_tok, d_out), jnp.float32),
        compiler_params=pltpu.CompilerParams(
            dimension_semantics=("parallel",)),
    )(x2, h, pmax, W_bf, A_bf, Bmb_bf, LiMEs)

    return out.reshape(B, T, d_out)


# speedup vs baseline: 1.0022x; 1.0022x over previous
import jax
import jax.numpy as jnp
from jax.experimental import pallas as pl
from jax.experimental.pallas import tpu as pltpu

E = 64
K = 8
R = 16
TEMP = 0.5
EPS = 1e-6
TILE = 1024


def _phase1_kernel(x_ref, ws_ref, h_ref, pmax_ref):
    h = jax.lax.dot_general(
        x_ref[:], ws_ref[:],
        dimension_numbers=(((1,), (1,)), ((), ())),
        preferred_element_type=jnp.float32,
    )
    h_ref[:] = h
    pmax_ref[:] = jnp.max(jnp.abs(h)).reshape(1, 1, 1)


def _phase2_kernel(x_ref, h_ref, pmax_ref, w_ref, a_ref, bmb_ref,
                   limes_ref, b_ref, out_ref):
    x = x_ref[:].astype(jnp.bfloat16)
    base = jax.lax.dot_general(
        x, w_ref[:],
        dimension_numbers=(((1,), (1,)), ((), ())),
        preferred_element_type=jnp.float32,
    )

    scale = jnp.maximum(jnp.max(pmax_ref[:]), EPS)
    inv = 1.0 / (scale * TEMP)
    logits = h_ref[:] * inv
    m = jnp.max(logits, axis=-1, keepdims=True)
    e = jnp.exp(logits - m)
    nch = 4
    ch = e.shape[0] // nch
    chunks = [e[i * ch:(i + 1) * ch] for i in range(nch)]
    for _ in range(K):
        curs = [jnp.max(c, axis=-1, keepdims=True) for c in chunks]
        chunks = [jnp.where(c == cur, -1.0, c)
                  for c, cur in zip(chunks, curs)]
    masked = jnp.concatenate(chunks, axis=0)
    sel_e = jnp.where(masked < 0.0, e, 0.0)
    ssum = jnp.sum(sel_e, axis=-1, keepdims=True)
    p_mix = jnp.dot(sel_e, limes_ref[:],
                    preferred_element_type=jnp.float32) / ssum

    u = jnp.dot(x, a_ref[:], preferred_element_type=jnp.float32)
    mod = (u * p_mix).astype(jnp.bfloat16)
    delta = jnp.dot(mod, bmb_ref[:],
                    preferred_element_type=jnp.float32)
    out_ref[:] = base + delta + b_ref[:]


def kernel(x, W, b, A, Bm, LiMEs):
    B, T, d_in = x.shape
    d_out = W.shape[0]
    n_tok = B * T
    nt = n_tok // TILE
    x2 = x.reshape(n_tok, d_in)
    W_bf = W.astype(jnp.bfloat16)
    A_bf = A.astype(jnp.bfloat16)
    Bm_bf = Bm.astype(jnp.bfloat16)

    h, pmax = pl.pallas_call(
        _phase1_kernel,
        grid=(nt,),
        in_specs=[
            pl.BlockSpec((TILE, d_in), lambda i: (i, 0)),
            pl.BlockSpec((E, d_in), lambda i: (0, 0)),
        ],
        out_specs=[
            pl.BlockSpec((TILE, E), lambda i: (i, 0)),
            pl.BlockSpec((1, 1, 1), lambda i: (i, 0, 0)),
        ],
        out_shape=[
            jax.ShapeDtypeStruct((n_tok, E), jnp.float32),
            jax.ShapeDtypeStruct((nt, 1, 1), jnp.float32),
        ],
        compiler_params=pltpu.CompilerParams(
            dimension_semantics=("parallel",)),
    )(x2, W)

    out = pl.pallas_call(
        _phase2_kernel,
        grid=(nt,),
        in_specs=[
            pl.BlockSpec((TILE, d_in), lambda i: (i, 0)),
            pl.BlockSpec((TILE, E), lambda i: (i, 0)),
            pl.BlockSpec((nt, 1, 1), lambda i: (0, 0, 0)),
            pl.BlockSpec((d_out, d_in), lambda i: (0, 0)),
            pl.BlockSpec((d_in, R), lambda i: (0, 0)),
            pl.BlockSpec((R, d_out), lambda i: (0, 0)),
            pl.BlockSpec((E, R), lambda i: (0, 0)),
            pl.BlockSpec((1, d_out), lambda i: (0, 0)),
        ],
        out_specs=pl.BlockSpec((TILE, d_out), lambda i: (i, 0)),
        out_shape=jax.ShapeDtypeStruct((n_tok, d_out), jnp.float32),
        compiler_params=pltpu.CompilerParams(
            dimension_semantics=("parallel",)),
    )(x2, h, pmax, W_bf, A_bf, Bm_bf, LiMEs, b.reshape(1, d_out))

    return out.reshape(B, T, d_out)


# speedup vs baseline: 1.1381x; 1.1356x over previous
import jax
import jax.numpy as jnp
from jax.experimental import pallas as pl
from jax.experimental.pallas import tpu as pltpu

E = 64
K = 8
R = 16
TEMP = 0.5
EPS = 1e-6
TILE = 1024
_FBASE = 0x3F800000


def _phase1_kernel(x_ref, ws_ref, h_ref, pmax_ref):
    h = jax.lax.dot_general(
        x_ref[:], ws_ref[:],
        dimension_numbers=(((1,), (1,)), ((), ())),
        preferred_element_type=jnp.float32,
    )
    h_ref[:] = h
    pmax_ref[:] = jnp.max(jnp.abs(h)).reshape(1, 1, 1)


def _phase2_kernel(x_ref, h_ref, pmax_ref, w_ref, b_ref, a_ref, bm_ref,
                   limes_ref, out_ref):
    x = x_ref[:].astype(jnp.bfloat16)
    base = jax.lax.dot_general(
        x, w_ref[:],
        dimension_numbers=(((1,), (1,)), ((), ())),
        preferred_element_type=jnp.float32,
    ) + b_ref[:]

    scale = jnp.maximum(jnp.max(pmax_ref[:]), EPS)
    inv = 1.0 / (scale * TEMP)
    logits = h_ref[:] * inv
    ii = jax.lax.broadcasted_iota(jnp.int32, logits.shape, 1)
    pbits = jax.lax.bitcast_convert_type(logits + 3.0, jnp.int32)
    keys = ((pbits - _FBASE) << 6) + (E - 1 - ii)

    kmax0 = jnp.max(keys, axis=-1, keepdims=True)
    mq = jax.lax.bitcast_convert_type(
        (kmax0 >> 6) + _FBASE, jnp.float32) - 3.0
    e = jnp.exp(logits - mq)

    wmat = jnp.zeros_like(e)
    masked = keys
    kmax = kmax0
    for k in range(K):
        if k:
            kmax = jnp.max(masked, axis=-1, keepdims=True)
        first = masked == kmax
        wmat = wmat + jnp.where(first, e, 0.0)
        masked = jnp.where(first, jnp.int32(-(2**31)), masked)

    ssum = jnp.sum(wmat, axis=-1, keepdims=True)
    w = wmat / ssum
    p_mix = jnp.dot(w, limes_ref[:], preferred_element_type=jnp.float32)

    u = jnp.dot(x, a_ref[:], preferred_element_type=jnp.float32)
    mod = (u * p_mix).astype(jnp.bfloat16)
    delta = jnp.dot(mod, bm_ref[:], preferred_element_type=jnp.float32)
    out_ref[:] = base + delta


def kernel(x, W, b, A, Bm, LiMEs):
    B, T, d_in = x.shape
    d_out = W.shape[0]
    n_tok = B * T
    nt = n_tok // TILE
    x2 = x.reshape(n_tok, d_in)
    W_bf = W.astype(jnp.bfloat16)
    A_bf = A.astype(jnp.bfloat16)
    Bm_bf = Bm.astype(jnp.bfloat16)

    h, pmax = pl.pallas_call(
        _phase1_kernel,
        grid=(nt,),
        in_specs=[
            pl.BlockSpec((TILE, d_in), lambda i: (i, 0)),
            pl.BlockSpec((E, d_in), lambda i: (0, 0)),
        ],
        out_specs=[
            pl.BlockSpec((TILE, E), lambda i: (i, 0)),
            pl.BlockSpec((1, 1, 1), lambda i: (i, 0, 0)),
        ],
        out_shape=[
            jax.ShapeDtypeStruct((n_tok, E), jnp.float32),
            jax.ShapeDtypeStruct((nt, 1, 1), jnp.float32),
        ],
        compiler_params=pltpu.CompilerParams(
            dimension_semantics=("parallel",)),
    )(x2, W)

    out = pl.pallas_call(
        _phase2_kernel,
        grid=(nt,),
        in_specs=[
            pl.BlockSpec((TILE, d_in), lambda i: (i, 0)),
            pl.BlockSpec((TILE, E), lambda i: (i, 0)),
            pl.BlockSpec((nt, 1, 1), lambda i: (0, 0, 0)),
            pl.BlockSpec((d_out, d_in), lambda i: (0, 0)),
            pl.BlockSpec((1, d_out), lambda i: (0, 0)),
            pl.BlockSpec((d_in, R), lambda i: (0, 0)),
            pl.BlockSpec((R, d_out), lambda i: (0, 0)),
            pl.BlockSpec((E, R), lambda i: (0, 0)),
        ],
        out_specs=pl.BlockSpec((TILE, d_out), lambda i: (i, 0)),
        out_shape=jax.ShapeDtypeStruct((n_tok, d_out), jnp.float32),
        compiler_params=pltpu.CompilerParams(
            dimension_semantics=("parallel",)),
    )(x2, h, pmax, W_bf, b.reshape(1, d_out), A_bf, Bm_bf, LiMEs)

    return out.reshape(B, T, d_out)


# speedup vs baseline: 1.1433x; 1.0046x over previous
import jax
import jax.numpy as jnp
from jax.experimental import pallas as pl
from jax.experimental.pallas import tpu as pltpu

E = 64
K = 8
R = 16
TEMP = 0.5
EPS = 1e-6
TILE = 1024
_FBASE = 0x3F800000


def _merged_kernel(x_ref, ws_ref, w_ref, b_ref, a_ref, bm_ref, limes_ref,
                   out_ref, h_scr, pmax_scr):
    p = pl.program_id(0)
    i = pl.program_id(1)

    @pl.when(p == 0)
    def _pass0():
        h = jax.lax.dot_general(
            x_ref[:], ws_ref[:],
            dimension_numbers=(((1,), (1,)), ((), ())),
            preferred_element_type=jnp.float32,
        )
        h_scr[pl.ds(i * TILE, TILE), :] = h
        tile_max = jnp.max(jnp.abs(h))

        @pl.when(i == 0)
        def _init():
            pmax_scr[0] = tile_max

        @pl.when(i != 0)
        def _acc():
            pmax_scr[0] = jnp.maximum(pmax_scr[0], tile_max)

    @pl.when(p == 1)
    def _pass1():
        x = x_ref[:].astype(jnp.bfloat16)
        base = jax.lax.dot_general(
            x, w_ref[:],
            dimension_numbers=(((1,), (1,)), ((), ())),
            preferred_element_type=jnp.float32,
        ) + b_ref[:]

        scale = jnp.maximum(pmax_scr[0], EPS)
        inv = 1.0 / (scale * TEMP)
        logits = h_scr[pl.ds(i * TILE, TILE), :] * inv
        ii = jax.lax.broadcasted_iota(jnp.int32, logits.shape, 1)
        pbits = jax.lax.bitcast_convert_type(logits + 3.0, jnp.int32)
        keys = ((pbits - _FBASE) << 6) + (E - 1 - ii)

        kmax0 = jnp.max(keys, axis=-1, keepdims=True)
        mq = jax.lax.bitcast_convert_type(
            (kmax0 >> 6) + _FBASE, jnp.float32) - 3.0
        e = jnp.exp(logits - mq)

        wmat = jnp.zeros_like(e)
        masked = keys
        kmax = kmax0
        for k in range(K):
            if k:
                kmax = jnp.max(masked, axis=-1, keepdims=True)
            first = masked == kmax
            wmat = wmat + jnp.where(first, e, 0.0)
            masked = jnp.where(first, jnp.int32(-(2**31)), masked)

        ssum = jnp.sum(wmat, axis=-1, keepdims=True)
        w = wmat / ssum
        p_mix = jnp.dot(w, limes_ref[:], preferred_element_type=jnp.float32)

        u = jnp.dot(x, a_ref[:], preferred_element_type=jnp.float32)
        mod = (u * p_mix).astype(jnp.bfloat16)
        delta = jnp.dot(mod, bm_ref[:], preferred_element_type=jnp.float32)
        out_ref[:] = base + delta


def kernel(x, W, b, A, Bm, LiMEs):
    B, T, d_in = x.shape
    d_out = W.shape[0]
    n_tok = B * T
    nt = n_tok // TILE
    x2 = x.reshape(n_tok, d_in)
    W_bf = W.astype(jnp.bfloat16)
    A_bf = A.astype(jnp.bfloat16)
    Bm_bf = Bm.astype(jnp.bfloat16)

    out = pl.pallas_call(
        _merged_kernel,
        grid=(2, nt),
        in_specs=[
            pl.BlockSpec((TILE, d_in), lambda p, i: (i, 0)),
            pl.BlockSpec((E, d_in), lambda p, i: (0, 0)),
            pl.BlockSpec((d_out, d_in), lambda p, i: (0, 0)),
            pl.BlockSpec((1, d_out), lambda p, i: (0, 0)),
            pl.BlockSpec((d_in, R), lambda p, i: (0, 0)),
            pl.BlockSpec((R, d_out), lambda p, i: (0, 0)),
            pl.BlockSpec((E, R), lambda p, i: (0, 0)),
        ],
        out_specs=pl.BlockSpec((TILE, d_out), lambda p, i: (p * i, 0)),
        out_shape=jax.ShapeDtypeStruct((n_tok, d_out), jnp.float32),
        scratch_shapes=[
            pltpu.VMEM((n_tok, E), jnp.float32),
            pltpu.SMEM((1,), jnp.float32),
        ],
        compiler_params=pltpu.CompilerParams(
            dimension_semantics=("arbitrary", "arbitrary")),
    )(x2, W, W_bf, b.reshape(1, d_out), A_bf, Bm_bf, LiMEs)

    return out.reshape(B, T, d_out)
